# baseline (speedup 1.0000x reference)
import jax
import jax.numpy as jnp
from jax import lax
from jax.experimental import pallas as pl
from jax.experimental.pallas import tpu as pltpu
from jax.experimental.pallas import tpu_sc as plsc

N_NODES = 10000
D = 128
NC = 2
NS = 16
CHUNK = 128
NBUF = 3

MESH_CHUNKS_PER_SC = 1250
MESH_ITERS = 79
WORLD_CHUNKS_PER_SC = 156
WORLD_ITERS = 10
WORLD_TAIL_BASE = 312 * CHUNK
WORLD_TAIL = 64

ROWS_MAIN = 624
TAIL_ROWS_BASE = NS * ROWS_MAIN
TAIL_ROWS = N_NODES - TAIL_ROWS_BASE


def _zero_rows(buf, n_rows):
    def zrow(i, _):
        for j in range(D // 16):
            buf[i, pl.ds(j * 16, 16)] = jnp.zeros((16,), jnp.float32)
        return _
    lax.fori_loop(0, n_rows, zrow, None)


def _sc_body(edge_attr, midx, wedge_attr, widx,
             out_mesh, out_world, acc,
             rows0, rows1, rows2, idx3,
             gsem0, gsem1, gsem2, ssem0, ssem1, ssem2):
    c = lax.axis_index("c")
    s = lax.axis_index("s")
    rows = (rows0, rows1, rows2)
    gsem = (gsem0, gsem1, gsem2)
    ssem = (ssem0, ssem1, ssem2)

    base = s * ROWS_MAIN

    def zero_own_slice():
        for k in range(4):
            pltpu.sync_copy(rows2, acc.at[pl.ds(base + k * CHUNK, CHUNK)])
        pltpu.sync_copy(rows2.at[pl.ds(0, ROWS_MAIN - 4 * CHUNK)],
                        acc.at[pl.ds(base + 4 * CHUNK,
                                     ROWS_MAIN - 4 * CHUNK)])

        @pl.when(s == NS - 1)
        def _():
            pltpu.sync_copy(rows2.at[pl.ds(0, TAIL_ROWS)],
                            acc.at[pl.ds(TAIL_ROWS_BASE, TAIL_ROWS)])

    def make_phase(n_iters, chunks_per_sc, idx_hbm, attr_hbm):
        def valid(t):
            return (t * NS + s) < chunks_per_sc

        def start_gather(t, b):
            @pl.when(valid(t))
            def _():
                chunk = c * chunks_per_sc + t * NS + s
                pltpu.async_copy(idx_hbm.at[pl.ds(chunk * CHUNK, CHUNK)],
                                 idx3.at[b], gsem[b])
                pltpu.async_copy(attr_hbm.at[pl.ds(chunk * CHUNK, CHUNK)],
                                 rows[b], gsem[b])

        def wait_gather(t, b):
            @pl.when(valid(t))
            def _():
                pltpu.make_async_copy(idx_hbm.at[pl.ds(0, CHUNK)],
                                      idx3.at[b], gsem[b]).wait()
                pltpu.make_async_copy(attr_hbm.at[pl.ds(0, CHUNK)],
                                      rows[b], gsem[b]).wait()

        def start_scatter(t, b):
            @pl.when(valid(t))
            def _():
                pltpu.async_copy(rows[b], acc.at[idx3.at[b]], ssem[b],
                                 add=True)

        n_outer = (n_iters + 2 + NBUF - 1) // NBUF + 1

        def prologue():
            start_gather(0, 0)
            start_gather(1, 1)

        def outer(jo, _):
            for b_ in range(NBUF):
                j = jo * NBUF + b_
                wait_gather(j, b_)
                start_scatter(j, b_)
                bprev = (b_ - 1) % NBUF
                jm1 = j - 1

                @pl.when(jnp.logical_and(jm1 >= 0, valid(jm1)))
                def _(bp=bprev):
                    pltpu.make_async_copy(rows[bp], acc.at[idx3.at[bp]],
                                          ssem[bp]).wait()

                start_gather(j + 2, bprev)
            return _

        def loop():
            lax.fori_loop(0, n_outer, outer, None)

        return prologue, loop

    def flush(out_hbm):
        pltpu.sync_copy(acc.at[pl.ds(base, ROWS_MAIN)],
                        out_hbm.at[c, pl.ds(base, ROWS_MAIN)])

        @pl.when(s == NS - 1)
        def _():
            pltpu.sync_copy(acc.at[pl.ds(TAIL_ROWS_BASE, TAIL_ROWS)],
                            out_hbm.at[c, pl.ds(TAIL_ROWS_BASE, TAIL_ROWS)])

    mesh_pro, mesh_loop = make_phase(MESH_ITERS, MESH_CHUNKS_PER_SC,
                                     midx, edge_attr)
    world_pro, world_loop = make_phase(WORLD_ITERS, WORLD_CHUNKS_PER_SC,
                                       widx, wedge_attr)

    mesh_pro()
    _zero_rows(rows2, CHUNK)
    zero_own_slice()
    plsc.subcore_barrier()
    mesh_loop()
    plsc.subcore_barrier()
    world_pro()
    flush(out_mesh)
    plsc.subcore_barrier()

    world_loop()

    @pl.when(jnp.logical_and(c == 0, s == 0))
    def _tail():
        for k in range((CHUNK - WORLD_TAIL) // 16):
            idx3[0, pl.ds(WORLD_TAIL + k * 16, 16)] = jnp.zeros(
                (16,), jnp.int32)
        pltpu.sync_copy(widx.at[pl.ds(WORLD_TAIL_BASE, WORLD_TAIL)],
                        idx3.at[0, pl.ds(0, WORLD_TAIL)])
        _zero_rows(rows0, CHUNK)
        pltpu.sync_copy(wedge_attr.at[pl.ds(WORLD_TAIL_BASE, WORLD_TAIL)],
                        rows0.at[pl.ds(0, WORLD_TAIL)])
        pltpu.sync_copy(rows0, acc.at[idx3.at[0]], add=True)

    plsc.subcore_barrier()
    flush(out_world)


def _sc_scatter(edge_attr, midx, wedge_attr, widx):
    mesh = plsc.VectorSubcoreMesh(core_axis_name="c", subcore_axis_name="s")
    f = pl.kernel(
        _sc_body,
        out_type=(
            jax.ShapeDtypeStruct((NC, N_NODES, D), jnp.float32),
            jax.ShapeDtypeStruct((NC, N_NODES, D), jnp.float32),
        ),
        mesh=mesh,
        scratch_types=[
            pltpu.VMEM_SHARED((N_NODES, D), jnp.float32),
            pltpu.VMEM((CHUNK, D), jnp.float32),
            pltpu.VMEM((CHUNK, D), jnp.float32),
            pltpu.VMEM((CHUNK, D), jnp.float32),
            pltpu.VMEM((NBUF, CHUNK), jnp.int32),
            pltpu.SemaphoreType.DMA,
            pltpu.SemaphoreType.DMA,
            pltpu.SemaphoreType.DMA,
            pltpu.SemaphoreType.DMA,
            pltpu.SemaphoreType.DMA,
            pltpu.SemaphoreType.DMA,
        ],
    )
    return f(edge_attr, midx, wedge_attr, widx)


def _pre_body(node_ref, w1_ref, b1_ref, out_ref):
    out_ref[...] = (jnp.dot(node_ref[...], w1_ref[0:D],
                            preferred_element_type=jnp.float32)
                    + b1_ref[...])


def _mlp_body(pre_ref, mesh_ref, world_ref, w1_ref, w2_ref, b2_ref,
              out_ref):
    m = mesh_ref[0] + mesh_ref[1]
    w = world_ref[0] + world_ref[1] - m
    h = (pre_ref[...]
         + jnp.dot(m, w1_ref[D:2 * D], preferred_element_type=jnp.float32)
         + jnp.dot(w, w1_ref[2 * D:3 * D], preferred_element_type=jnp.float32))
    h = jnp.maximum(h, 0.0)
    out_ref[...] = (jnp.dot(h, w2_ref[...], preferred_element_type=jnp.float32)
                    + b2_ref[...])


def _tc_pre(node_attr, W1, b1):
    R = 2000
    return pl.pallas_call(
        _pre_body,
        grid=(N_NODES // R,),
        in_specs=[
            pl.BlockSpec((R, D), lambda i: (i, 0)),
            pl.BlockSpec((3 * D, D), lambda i: (0, 0)),
            pl.BlockSpec((1, D), lambda i: (0, 0)),
        ],
        out_specs=pl.BlockSpec((R, D), lambda i: (i, 0)),
        out_shape=jax.ShapeDtypeStruct((N_NODES, D), jnp.float32),
    )(node_attr, W1, b1)


def _tc_mlp(pre, mesh_p, world_p, W1, W2, b2):
    R = 2000
    grid = (N_NODES // R,)
    return pl.pallas_call(
        _mlp_body,
        grid=grid,
        in_specs=[
            pl.BlockSpec((R, D), lambda i: (i, 0)),
            pl.BlockSpec((NC, R, D), lambda i: (0, i, 0)),
            pl.BlockSpec((NC, R, D), lambda i: (0, i, 0)),
            pl.BlockSpec((3 * D, D), lambda i: (0, 0)),
            pl.BlockSpec((D, D), lambda i: (0, 0)),
            pl.BlockSpec((1, D), lambda i: (0, 0)),
        ],
        out_specs=pl.BlockSpec((R, D), lambda i: (i, 0)),
        out_shape=jax.ShapeDtypeStruct((N_NODES, D), jnp.float32),
    )(pre, mesh_p, world_p, W1, W2, b2)


def kernel(node_attr, edge_index, edge_attr, edge_world_index,
           edge_world_attr, W1, b1, W2, b2):
    midx = edge_index[1].astype(jnp.int32)
    widx = edge_world_index[1].astype(jnp.int32)
    pre = _tc_pre(node_attr, W1, b1.reshape(1, D))
    mesh_p, world_p = _sc_scatter(edge_attr, midx, edge_world_attr, widx)
    return _tc_mlp(pre, mesh_p, world_p, W1, W2, b2.reshape(1, D))


# speedup vs baseline: 1.0084x; 1.0084x over previous
import jax
import jax.numpy as jnp
from jax import lax
from jax.experimental import pallas as pl
from jax.experimental.pallas import tpu as pltpu
from jax.experimental.pallas import tpu_sc as plsc

N_NODES = 10000
D = 128
NC = 2
NS = 16
CHUNK = 128
NBUF = 3

MESH_CHUNKS_PER_SC = 1250
MESH_ITERS = 79
WORLD_CHUNKS_PER_SC = 156
WORLD_ITERS = 10
WORLD_TAIL_BASE = 312 * CHUNK
WORLD_TAIL = 64

ROWS_MAIN = 624
TAIL_ROWS_BASE = NS * ROWS_MAIN
TAIL_ROWS = N_NODES - TAIL_ROWS_BASE


def _zero_rows(buf, n_rows):
    def zrow(i, _):
        for j in range(D // 16):
            buf[i, pl.ds(j * 16, 16)] = jnp.zeros((16,), jnp.float32)
        return _
    lax.fori_loop(0, n_rows, zrow, None)


def _sc_body(edge_attr, midx, wedge_attr, widx,
             out_mesh, out_world, acc,
             rows0, rows1, rows2, idx3,
             gsem0, gsem1, gsem2, ssem0, ssem1, ssem2):
    c = lax.axis_index("c")
    s = lax.axis_index("s")
    rows = (rows0, rows1, rows2)
    gsem = (gsem0, gsem1, gsem2)
    ssem = (ssem0, ssem1, ssem2)

    base = s * ROWS_MAIN

    def zero_own_slice():
        for k in range(4):
            pltpu.sync_copy(rows2, acc.at[pl.ds(base + k * CHUNK, CHUNK)])
        pltpu.sync_copy(rows2.at[pl.ds(0, ROWS_MAIN - 4 * CHUNK)],
                        acc.at[pl.ds(base + 4 * CHUNK,
                                     ROWS_MAIN - 4 * CHUNK)])

        @pl.when(s == NS - 1)
        def _():
            pltpu.sync_copy(rows2.at[pl.ds(0, TAIL_ROWS)],
                            acc.at[pl.ds(TAIL_ROWS_BASE, TAIL_ROWS)])

    def make_phase(n_iters, chunks_per_sc, idx_hbm, attr_hbm):
        def valid(t):
            return (t * NS + s) < chunks_per_sc

        def start_gather(t, b):
            @pl.when(valid(t))
            def _():
                chunk = c * chunks_per_sc + t * NS + s
                pltpu.async_copy(idx_hbm.at[pl.ds(chunk * CHUNK, CHUNK)],
                                 idx3.at[b], gsem[b])
                pltpu.async_copy(attr_hbm.at[pl.ds(chunk * CHUNK, CHUNK)],
                                 rows[b], gsem[b])

        def wait_gather(t, b):
            @pl.when(valid(t))
            def _():
                pltpu.make_async_copy(idx_hbm.at[pl.ds(0, CHUNK)],
                                      idx3.at[b], gsem[b]).wait()
                pltpu.make_async_copy(attr_hbm.at[pl.ds(0, CHUNK)],
                                      rows[b], gsem[b]).wait()

        def start_scatter(t, b):
            @pl.when(valid(t))
            def _():
                pltpu.async_copy(rows[b], acc.at[idx3.at[b]], ssem[b],
                                 add=True)

        n_outer = (n_iters + 2 + NBUF - 1) // NBUF + 1

        def prologue():
            start_gather(0, 0)
            start_gather(1, 1)

        def outer(jo, _):
            for b_ in range(NBUF):
                j = jo * NBUF + b_
                wait_gather(j, b_)
                start_scatter(j, b_)
                bprev = (b_ - 1) % NBUF
                jm1 = j - 1

                @pl.when(jnp.logical_and(jm1 >= 0, valid(jm1)))
                def _(bp=bprev):
                    pltpu.make_async_copy(rows[bp], acc.at[idx3.at[bp]],
                                          ssem[bp]).wait()

                start_gather(j + 2, bprev)
            return _

        def loop():
            lax.fori_loop(0, n_outer, outer, None)

        return prologue, loop

    def flush(out_hbm):
        pltpu.sync_copy(acc.at[pl.ds(base, ROWS_MAIN)],
                        out_hbm.at[c, pl.ds(base, ROWS_MAIN)])

        @pl.when(s == NS - 1)
        def _():
            pltpu.sync_copy(acc.at[pl.ds(TAIL_ROWS_BASE, TAIL_ROWS)],
                            out_hbm.at[c, pl.ds(TAIL_ROWS_BASE, TAIL_ROWS)])

    mesh_pro, mesh_loop = make_phase(MESH_ITERS, MESH_CHUNKS_PER_SC,
                                     midx, edge_attr)
    world_pro, world_loop = make_phase(WORLD_ITERS, WORLD_CHUNKS_PER_SC,
                                       widx, wedge_attr)

    mesh_pro()
    _zero_rows(rows2, CHUNK)
    zero_own_slice()
    plsc.subcore_barrier()
    mesh_loop()
    plsc.subcore_barrier()
    world_pro()
    flush(out_mesh)
    plsc.subcore_barrier()

    world_loop()

    @pl.when(jnp.logical_and(c == 0, s == 0))
    def _tail():
        for k in range((CHUNK - WORLD_TAIL) // 16):
            idx3[0, pl.ds(WORLD_TAIL + k * 16, 16)] = jnp.zeros(
                (16,), jnp.int32)
        pltpu.sync_copy(widx.at[pl.ds(WORLD_TAIL_BASE, WORLD_TAIL)],
                        idx3.at[0, pl.ds(0, WORLD_TAIL)])
        _zero_rows(rows0, CHUNK)
        pltpu.sync_copy(wedge_attr.at[pl.ds(WORLD_TAIL_BASE, WORLD_TAIL)],
                        rows0.at[pl.ds(0, WORLD_TAIL)])
        pltpu.sync_copy(rows0, acc.at[idx3.at[0]], add=True)

    plsc.subcore_barrier()
    flush(out_world)


def _sc_scatter(edge_attr, midx, wedge_attr, widx):
    mesh = plsc.VectorSubcoreMesh(core_axis_name="c", subcore_axis_name="s")
    f = pl.kernel(
        _sc_body,
        out_type=(
            jax.ShapeDtypeStruct((NC, N_NODES, D), jnp.float32),
            jax.ShapeDtypeStruct((NC, N_NODES, D), jnp.float32),
        ),
        mesh=mesh,
        scratch_types=[
            pltpu.VMEM_SHARED((N_NODES, D), jnp.float32),
            pltpu.VMEM((CHUNK, D), jnp.float32),
            pltpu.VMEM((CHUNK, D), jnp.float32),
            pltpu.VMEM((CHUNK, D), jnp.float32),
            pltpu.VMEM((NBUF, CHUNK), jnp.int32),
            pltpu.SemaphoreType.DMA,
            pltpu.SemaphoreType.DMA,
            pltpu.SemaphoreType.DMA,
            pltpu.SemaphoreType.DMA,
            pltpu.SemaphoreType.DMA,
            pltpu.SemaphoreType.DMA,
        ],
    )
    return f(edge_attr, midx, wedge_attr, widx)


def _mlp_body(node_ref, mesh_ref, world_ref, w1_ref, b1_ref, w2_ref, b2_ref,
              out_ref):
    x = node_ref[...]
    m = mesh_ref[0] + mesh_ref[1]
    w = world_ref[0] + world_ref[1] - m
    h = (jnp.dot(x, w1_ref[0:D], preferred_element_type=jnp.float32)
         + jnp.dot(m, w1_ref[D:2 * D], preferred_element_type=jnp.float32)
         + jnp.dot(w, w1_ref[2 * D:3 * D], preferred_element_type=jnp.float32)
         + b1_ref[...])
    h = jnp.maximum(h, 0.0)
    out_ref[...] = (jnp.dot(h, w2_ref[...], preferred_element_type=jnp.float32)
                    + b2_ref[...])


def _tc_mlp(node_attr, mesh_p, world_p, W1, b1, W2, b2):
    R = 2000
    grid = (N_NODES // R,)
    return pl.pallas_call(
        _mlp_body,
        grid=grid,
        in_specs=[
            pl.BlockSpec((R, D), lambda i: (i, 0)),
            pl.BlockSpec((NC, R, D), lambda i: (0, i, 0)),
            pl.BlockSpec((NC, R, D), lambda i: (0, i, 0)),
            pl.BlockSpec((3 * D, D), lambda i: (0, 0)),
            pl.BlockSpec((1, D), lambda i: (0, 0)),
            pl.BlockSpec((D, D), lambda i: (0, 0)),
            pl.BlockSpec((1, D), lambda i: (0, 0)),
        ],
        out_specs=pl.BlockSpec((R, D), lambda i: (i, 0)),
        out_shape=jax.ShapeDtypeStruct((N_NODES, D), jnp.float32),
    )(node_attr, mesh_p, world_p, W1, b1, W2, b2)


def kernel(node_attr, edge_index, edge_attr, edge_world_index,
           edge_world_attr, W1, b1, W2, b2):
    midx = edge_index[1].astype(jnp.int32)
    widx = edge_world_index[1].astype(jnp.int32)
    mesh_p, world_p = _sc_scatter(edge_attr, midx, edge_world_attr, widx)
    return _tc_mlp(node_attr, mesh_p, world_p, W1,
                   b1.reshape(1, D), W2, b2.reshape(1, D))
